# baseline (speedup 1.0000x reference)
import jax
import jax.numpy as jnp
from jax import lax
from jax.experimental import pallas as pl
from jax.experimental.pallas import tpu as pltpu

_BLOCK = 5000


def _linear_kernel(x_ref, w_ref, b_ref, o_ref):
    xb = x_ref[...]
    w = w_ref[...]
    dims = (((1,), (1,)), ((), ()))
    lo = lax.dot_general(xb[:, :64], w, dims, preferred_element_type=jnp.float32)
    hi = lax.dot_general(xb[:, 64:], w, dims, preferred_element_type=jnp.float32)
    o_ref[...] = jnp.concatenate([lo, hi], axis=1) + b_ref[...]


def kernel(x, edge_index, W, b):
    n, h = x.shape
    out_dim = W.shape[0]
    x2 = x.reshape(n // 2, 2 * h)
    b2 = jnp.concatenate([b, b]).reshape(1, 2 * out_dim)
    out2 = pl.pallas_call(
        _linear_kernel,
        grid=(x2.shape[0] // _BLOCK,),
        in_specs=[
            pl.BlockSpec((_BLOCK, 2 * h), lambda i: (i, 0)),
            pl.BlockSpec((out_dim, h), lambda i: (0, 0)),
            pl.BlockSpec((1, 2 * out_dim), lambda i: (0, 0)),
        ],
        out_specs=pl.BlockSpec((_BLOCK, 2 * out_dim), lambda i: (i, 0)),
        out_shape=jax.ShapeDtypeStruct((n // 2, 2 * out_dim), jnp.float32),
        compiler_params=pltpu.CompilerParams(
            dimension_semantics=("parallel",),
        ),
    )(x2, W, b2)
    return out2.reshape(n, out_dim)


# speedup vs baseline: 1.7717x; 1.7717x over previous
import jax
import jax.numpy as jnp
from jax import lax
from jax.experimental import pallas as pl
from jax.experimental.pallas import tpu as pltpu

_BLOCK = 10000


def _linear_kernel(x_ref, w_ref, b_ref, o_ref):
    o_ref[...] = lax.dot_general(
        x_ref[...], w_ref[...],
        (((1,), (1,)), ((), ())),
        preferred_element_type=jnp.float32,
    ) + b_ref[...]


def kernel(x, edge_index, W, b):
    n, h = x.shape
    out_dim = W.shape[0]
    b2 = b.reshape(1, out_dim)
    return pl.pallas_call(
        _linear_kernel,
        grid=(n // _BLOCK,),
        in_specs=[
            pl.BlockSpec((_BLOCK, h), lambda i: (i, 0)),
            pl.BlockSpec((out_dim, h), lambda i: (0, 0)),
            pl.BlockSpec((1, out_dim), lambda i: (0, 0)),
        ],
        out_specs=pl.BlockSpec((_BLOCK, out_dim), lambda i: (i, 0)),
        out_shape=jax.ShapeDtypeStruct((n, out_dim), jnp.float32),
        compiler_params=pltpu.CompilerParams(
            dimension_semantics=("parallel",),
        ),
    )(x, W, b2)
